# baseline (speedup 1.0000x reference)
import functools

import jax
import jax.numpy as jnp
import numpy as np
from jax import lax
from jax.experimental import pallas as pl
from jax.experimental.pallas import tpu as pltpu
from jax.experimental.pallas import tpu_sc as plsc

_NC = 2
_NS = 16
_NW = _NC * _NS
_W = 128
_G = 64


def _sc_mesh():
    return plsc.VectorSubcoreMesh(core_axis_name="c", subcore_axis_name="s")


def _row_split(n):
    assert n % 8 == 0
    per = (n // (8 * _NS)) * 8
    rows = [per] * _NS
    rows[-1] += n - per * _NS
    starts = [sum(rows[:i]) for i in range(_NS)]
    return rows, starts


def _sc_deg(dst, n):
    e = dst.shape[0]
    per_w = e // _NW
    full = per_w // _W
    tail = per_w - full * _W
    assert full % 2 == 0, "paired prefetch needs an even full-window count"
    n_pad = ((n + 127) // 128) * 128
    blocks = n_pad // 128
    blk_per = [blocks // _NS + (1 if i < blocks % _NS else 0)
               for i in range(_NS)]
    blk_start = [sum(blk_per[:i]) for i in range(_NS)]

    @functools.partial(
        pl.kernel,
        out_type=jax.ShapeDtypeStruct((_NC, n_pad), jnp.float32),
        mesh=_sc_mesh(),
        scratch_types=[
            pltpu.VMEM((1, _W), jnp.int32),
            pltpu.VMEM((1, _W), jnp.int32),
            pltpu.VMEM((1, 16), jnp.int32),
            pltpu.VMEM((_W,), jnp.float32),
            pltpu.VMEM((_W,), jnp.float32),
            pltpu.VMEM((16,), jnp.float32),
            pltpu.VMEM_SHARED((n_pad,), jnp.float32),
            pltpu.SemaphoreType.DMA,
            pltpu.SemaphoreType.DMA,
            pltpu.SemaphoreType.DMA,
            pltpu.SemaphoreType.DMA,
        ],
    )
    def deg_kernel(dst_hbm, out_hbm, didx_a, didx_b, dtail, zbuf, obuf,
                   otail, acc, sem_a, sem_b, sem_sa, sem_sb):
        c = lax.axis_index("c")
        s = lax.axis_index("s")
        wid = c * _NS + s

        zero_pat = jnp.zeros((16,), jnp.float32)
        one_pat = jnp.full((16,), 1.0, jnp.float32)

        @pl.loop(0, _W, step=16)
        def _(i):
            zbuf[pl.ds(i, 16)] = zero_pat
            obuf[pl.ds(i, 16)] = one_pat
        otail[...] = one_pat

        for sub in range(_NS):
            @pl.when(s == sub)
            def _():
                for k in range(blk_per[sub]):
                    off = (blk_start[sub] + k) * 128
                    pltpu.sync_copy(zbuf, acc.at[pl.ds(off, 128)])

        plsc.subcore_barrier()

        def start_load(win, dref, sem):
            pltpu.async_copy(dst_hbm.at[pl.ds(wid * per_w + win * _W, _W)],
                             dref.at[0], sem)

        def wait_load(win, dref, sem):
            pltpu.make_async_copy(
                dst_hbm.at[pl.ds(wid * per_w + win * _W, _W)],
                dref.at[0], sem).wait()

        start_load(0, didx_a, sem_a)
        start_load(1, didx_b, sem_b)

        @pl.loop(0, full, step=2)
        def _(i):
            wait_load(i, didx_a, sem_a)
            pltpu.async_copy(obuf, acc.at[didx_a.at[0]], sem_sa, add=True)
            wait_load(i + 1, didx_b, sem_b)
            pltpu.async_copy(obuf, acc.at[didx_b.at[0]], sem_sb, add=True)

            @pl.when(i + 2 < full)
            def _():
                pltpu.make_async_copy(obuf, acc.at[didx_a.at[0]],
                                      sem_sa).wait()
                start_load(i + 2, didx_a, sem_a)

            @pl.when(i + 3 < full)
            def _():
                pltpu.make_async_copy(obuf, acc.at[didx_b.at[0]],
                                      sem_sb).wait()
                start_load(i + 3, didx_b, sem_b)

        pltpu.make_async_copy(obuf, acc.at[didx_a.at[0]], sem_sa).wait()
        pltpu.make_async_copy(obuf, acc.at[didx_b.at[0]], sem_sb).wait()

        if tail:
            pltpu.sync_copy(dst_hbm.at[pl.ds(wid * per_w + full * _W, tail)],
                            dtail.at[0])
            pltpu.sync_copy(otail, acc.at[dtail.at[0]], add=True)

        plsc.subcore_barrier()
        for sub in range(_NS):
            @pl.when(s == sub)
            def _():
                off = blk_start[sub] * 128
                nrow = blk_per[sub] * 128
                pltpu.sync_copy(acc.at[pl.ds(off, nrow)],
                                out_hbm.at[c].at[pl.ds(off, nrow)])

    return deg_kernel(dst)


def _sc_gather_scatter(y, src, dst):
    n, d = y.shape
    e = src.shape[0]
    per_w = e // _NW
    full = per_w // _W
    tail = per_w - full * _W
    assert full % 2 == 0, "double buffering needs an even full-window count"
    rows_per_sub, row_start = _row_split(n)

    @functools.partial(
        pl.kernel,
        out_type=jax.ShapeDtypeStruct((_NC, n, d), jnp.float32),
        mesh=_sc_mesh(),
        scratch_types=[
            pltpu.VMEM((1, _W), jnp.int32),
            pltpu.VMEM((1, _W), jnp.int32),
            pltpu.VMEM((1, _W), jnp.int32),
            pltpu.VMEM((1, _W), jnp.int32),
            pltpu.VMEM((1, _W), jnp.int32),
            pltpu.VMEM((1, _W), jnp.int32),
            pltpu.VMEM((1, 16), jnp.int32),
            pltpu.VMEM((1, 16), jnp.int32),
            pltpu.VMEM((_W, d), jnp.float32),
            pltpu.VMEM((_W, d), jnp.float32),
            pltpu.VMEM_SHARED((n, d), jnp.float32),
            pltpu.SemaphoreType.DMA,
            pltpu.SemaphoreType.DMA,
            pltpu.SemaphoreType.DMA,
            pltpu.SemaphoreType.DMA,
            pltpu.SemaphoreType.DMA,
            pltpu.SemaphoreType.DMA,
        ],
    )
    def gs_kernel(y_hbm, src_hbm, dst_hbm, out_hbm,
                  sa, sb, da, db, da2, db2, st, dt, rows_a, rows_b, acc,
                  sem_a, sem_b, sem_ia, sem_ib, sem_sa, sem_sb):
        c = lax.axis_index("c")
        s = lax.axis_index("s")
        wid = c * _NS + s
        base_e = wid * per_w

        zero_pat = jnp.zeros((16,), jnp.float32)

        @pl.loop(0, _W)
        def _(i):
            @pl.loop(0, d, step=16)
            def _(j):
                rows_a[i, pl.ds(j, 16)] = zero_pat

        for sub in range(_NS):
            @pl.when(s == sub)
            def _():
                base = row_start[sub]
                nrow = rows_per_sub[sub]
                nfull = nrow // _W
                for k in range(nfull):
                    pltpu.sync_copy(rows_a.at[pl.ds(0, _W)],
                                    acc.at[pl.ds(base + k * _W, _W)])
                rem = nrow - nfull * _W
                if rem:
                    pltpu.sync_copy(rows_a.at[pl.ds(0, rem)],
                                    acc.at[pl.ds(base + nfull * _W, rem)])

        plsc.subcore_barrier()

        def start_idx(win, sref, dref, sem):
            off = base_e + win * _W
            pltpu.async_copy(src_hbm.at[pl.ds(off, _W)], sref.at[0], sem)
            pltpu.async_copy(dst_hbm.at[pl.ds(off, _W)], dref.at[0], sem)

        def wait_idx(win, sref, dref, sem):
            off = base_e + win * _W
            pltpu.make_async_copy(src_hbm.at[pl.ds(off, _W)],
                                  sref.at[0], sem).wait()
            pltpu.make_async_copy(dst_hbm.at[pl.ds(off, _W)],
                                  dref.at[0], sem).wait()

        start_idx(0, sa, da, sem_ia)
        start_idx(1, sb, db, sem_ib)
        wait_idx(0, sa, da, sem_ia)
        pltpu.async_copy(y_hbm.at[sa.at[0]], rows_a, sem_a)

        def idx_copy(dref, d2ref):
            @pl.loop(0, _W, step=16)
            def _(j):
                d2ref[0, pl.ds(j, 16)] = dref[0, pl.ds(j, 16)]

        @pl.loop(0, full, step=2)
        def _(i):
            pltpu.make_async_copy(y_hbm.at[sa.at[0]], rows_a, sem_a).wait()
            wait_idx(i + 1, sb, db, sem_ib)
            pltpu.async_copy(y_hbm.at[sb.at[0]], rows_b, sem_b)
            idx_copy(da, da2)
            pltpu.async_copy(rows_a, acc.at[da2.at[0]], sem_sa, add=True)

            @pl.when(i + 2 < full)
            def _():
                start_idx(i + 2, sa, da, sem_ia)

            pltpu.make_async_copy(y_hbm.at[sb.at[0]], rows_b, sem_b).wait()
            idx_copy(db, db2)
            pltpu.async_copy(rows_b, acc.at[db2.at[0]], sem_sb, add=True)

            @pl.when(i + 3 < full)
            def _():
                start_idx(i + 3, sb, db, sem_ib)

            pltpu.make_async_copy(rows_a, acc.at[da2.at[0]], sem_sa).wait()

            @pl.when(i + 2 < full)
            def _():
                wait_idx(i + 2, sa, da, sem_ia)
                pltpu.async_copy(y_hbm.at[sa.at[0]], rows_a, sem_a)

            pltpu.make_async_copy(rows_b, acc.at[db2.at[0]], sem_sb).wait()

        if tail:
            off = base_e + full * _W
            pltpu.sync_copy(src_hbm.at[pl.ds(off, tail)], st.at[0])
            pltpu.sync_copy(dst_hbm.at[pl.ds(off, tail)], dt.at[0])
            pltpu.sync_copy(y_hbm.at[st.at[0]], rows_a.at[pl.ds(0, tail)])
            pltpu.sync_copy(rows_a.at[pl.ds(0, tail)],
                            acc.at[dt.at[0]], add=True)

        plsc.subcore_barrier()
        for sub in range(_NS):
            @pl.when(s == sub)
            def _():
                base = row_start[sub]
                nrow = rows_per_sub[sub]
                pltpu.sync_copy(acc.at[pl.ds(base, nrow)],
                                out_hbm.at[c].at[pl.ds(base, nrow)])

    return gs_kernel(y, src, dst)


def _mm_body(x_ref, w_ref, o_ref):
    o_ref[...] = jnp.dot(x_ref[...], w_ref[...],
                         preferred_element_type=jnp.float32)


def _tc_matmul(x, w):
    n = x.shape[0]
    return pl.pallas_call(
        _mm_body,
        out_shape=jax.ShapeDtypeStruct((n, w.shape[1]), jnp.float32),
    )(x, w)


def _scale_body(d0_ref, d1_ref, xw_ref, y_ref, dinv_ref):
    deg = d0_ref[...] + d1_ref[...] + 1.0
    dinv = lax.rsqrt(deg)
    dinv_ref[...] = dinv
    y_ref[...] = xw_ref[...] * dinv


def _tc_scale(d0, d1, xw):
    n, d = xw.shape
    return pl.pallas_call(
        _scale_body,
        out_shape=(jax.ShapeDtypeStruct((n, d), jnp.float32),
                   jax.ShapeDtypeStruct((n, 1), jnp.float32)),
    )(d0, d1, xw)


def _mid_body(acc_ref, y1_ref, dinv_ref, b1_ref, w2_ref, y2_ref):
    dinv = dinv_ref[...]
    out1 = (acc_ref[0] + acc_ref[1] + y1_ref[...]) * dinv + b1_ref[...]
    h = 0.5 * out1 * (1.0 + lax.erf(out1 * np.float32(0.7071067811865476)))
    y2_ref[...] = jnp.dot(h, w2_ref[...],
                          preferred_element_type=jnp.float32) * dinv


def _tc_mid(acc1, y1, dinv, b1, w2):
    n = y1.shape[0]
    return pl.pallas_call(
        _mid_body,
        out_shape=jax.ShapeDtypeStruct((n, w2.shape[1]), jnp.float32),
    )(acc1, y1, dinv, b1, w2)


def _final_body(acc_ref, y2_ref, dinv_ref, b2_ref, batch_ref, fcw_ref,
                fcb_ref, o_ref):
    n = y2_ref.shape[0]
    out2 = (acc_ref[0] + acc_ref[1] + y2_ref[...]) * dinv_ref[...] \
        + b2_ref[...]
    gid = lax.broadcasted_iota(jnp.int32, (1, _G), 1)
    seg = (batch_ref[...] == gid).astype(jnp.float32)
    dn = (((0,), (0,)), ((), ()))
    psum = lax.dot_general(seg, out2, dn,
                           preferred_element_type=jnp.float32)
    ones = jnp.ones((n, 1), jnp.float32)
    cnt = lax.dot_general(seg, ones, dn,
                          preferred_element_type=jnp.float32)
    pooled = psum / jnp.maximum(cnt, 1.0)
    o_ref[...] = jnp.dot(pooled, fcw_ref[...],
                         preferred_element_type=jnp.float32) + fcb_ref[...]


def _tc_final(acc2, y2, dinv, b2, batch2d, fcW, fcb):
    return pl.pallas_call(
        _final_body,
        out_shape=jax.ShapeDtypeStruct((_G, fcW.shape[1]), jnp.float32),
    )(acc2, y2, dinv, b2, batch2d, fcW, fcb)


def kernel(x, edge_index, batch, W1, b1, W2, b2, fcW, fcb):
    n = x.shape[0]
    e = edge_index.shape[1]
    assert e % (_NW * 8) == 0, "edge count must split 8-aligned across workers"

    src = edge_index[0]
    dst = edge_index[1]
    batch2d = batch.reshape(n, 1)

    xw1 = _tc_matmul(x, W1)
    deg_p = _sc_deg(dst, n)
    d0 = deg_p[0, :n].reshape(n, 1)
    d1 = deg_p[1, :n].reshape(n, 1)
    y1, dinv = _tc_scale(d0, d1, xw1)
    acc1 = _sc_gather_scatter(y1, src, dst)
    y2 = _tc_mid(acc1, y1, dinv, b1, W2)
    acc2 = _sc_gather_scatter(y2, src, dst)
    return _tc_final(acc2, y2, dinv, b2, batch2d, fcW, fcb)


# speedup vs baseline: 1.0420x; 1.0420x over previous
import functools

import jax
import jax.numpy as jnp
import numpy as np
from jax import lax
from jax.experimental import pallas as pl
from jax.experimental.pallas import tpu as pltpu
from jax.experimental.pallas import tpu_sc as plsc

_NC = 2
_NS = 16
_NW = _NC * _NS
_W = 128
_G = 64


def _sc_mesh():
    return plsc.VectorSubcoreMesh(core_axis_name="c", subcore_axis_name="s")


def _row_split(n):
    assert n % 8 == 0
    per = (n // (8 * _NS)) * 8
    rows = [per] * _NS
    rows[-1] += n - per * _NS
    starts = [sum(rows[:i]) for i in range(_NS)]
    return rows, starts


def _sc_deg(eidx, e, n):
    per_w = e // _NW
    full = per_w // _W
    tail = per_w - full * _W
    assert full % 2 == 0, "paired prefetch needs an even full-window count"
    n_pad = ((n + 127) // 128) * 128
    blocks = n_pad // 128
    blk_per = [blocks // _NS + (1 if i < blocks % _NS else 0)
               for i in range(_NS)]
    blk_start = [sum(blk_per[:i]) for i in range(_NS)]

    @functools.partial(
        pl.kernel,
        out_type=jax.ShapeDtypeStruct((_NC, n_pad), jnp.float32),
        mesh=_sc_mesh(),
        scratch_types=[
            pltpu.VMEM((1, _W), jnp.int32),
            pltpu.VMEM((1, _W), jnp.int32),
            pltpu.VMEM((1, 16), jnp.int32),
            pltpu.VMEM((_W,), jnp.float32),
            pltpu.VMEM((_W,), jnp.float32),
            pltpu.VMEM((16,), jnp.float32),
            pltpu.VMEM_SHARED((n_pad,), jnp.float32),
            pltpu.SemaphoreType.DMA,
            pltpu.SemaphoreType.DMA,
            pltpu.SemaphoreType.DMA,
            pltpu.SemaphoreType.DMA,
        ],
    )
    def deg_kernel(dst_hbm, out_hbm, didx_a, didx_b, dtail, zbuf, obuf,
                   otail, acc, sem_a, sem_b, sem_sa, sem_sb):
        c = lax.axis_index("c")
        s = lax.axis_index("s")
        wid = c * _NS + s

        zero_pat = jnp.zeros((16,), jnp.float32)
        one_pat = jnp.full((16,), 1.0, jnp.float32)

        @pl.loop(0, _W, step=16)
        def _(i):
            zbuf[pl.ds(i, 16)] = zero_pat
            obuf[pl.ds(i, 16)] = one_pat
        otail[...] = one_pat

        for sub in range(_NS):
            @pl.when(s == sub)
            def _():
                for k in range(blk_per[sub]):
                    off = (blk_start[sub] + k) * 128
                    pltpu.sync_copy(zbuf, acc.at[pl.ds(off, 128)])

        plsc.subcore_barrier()

        def start_load(win, dref, sem):
            pltpu.async_copy(
                dst_hbm.at[pl.ds(e + wid * per_w + win * _W, _W)],
                dref.at[0], sem)

        def wait_load(win, dref, sem):
            pltpu.make_async_copy(
                dst_hbm.at[pl.ds(e + wid * per_w + win * _W, _W)],
                dref.at[0], sem).wait()

        start_load(0, didx_a, sem_a)
        start_load(1, didx_b, sem_b)

        @pl.loop(0, full, step=2)
        def _(i):
            wait_load(i, didx_a, sem_a)
            pltpu.async_copy(obuf, acc.at[didx_a.at[0]], sem_sa, add=True)
            wait_load(i + 1, didx_b, sem_b)
            pltpu.async_copy(obuf, acc.at[didx_b.at[0]], sem_sb, add=True)

            @pl.when(i + 2 < full)
            def _():
                pltpu.make_async_copy(obuf, acc.at[didx_a.at[0]],
                                      sem_sa).wait()
                start_load(i + 2, didx_a, sem_a)

            @pl.when(i + 3 < full)
            def _():
                pltpu.make_async_copy(obuf, acc.at[didx_b.at[0]],
                                      sem_sb).wait()
                start_load(i + 3, didx_b, sem_b)

        pltpu.make_async_copy(obuf, acc.at[didx_a.at[0]], sem_sa).wait()
        pltpu.make_async_copy(obuf, acc.at[didx_b.at[0]], sem_sb).wait()

        if tail:
            pltpu.sync_copy(
                dst_hbm.at[pl.ds(e + wid * per_w + full * _W, tail)],
                dtail.at[0])
            pltpu.sync_copy(otail, acc.at[dtail.at[0]], add=True)

        plsc.subcore_barrier()
        for sub in range(_NS):
            @pl.when(s == sub)
            def _():
                off = blk_start[sub] * 128
                nrow = blk_per[sub] * 128
                pltpu.sync_copy(acc.at[pl.ds(off, nrow)],
                                out_hbm.at[c].at[pl.ds(off, nrow)])

    return deg_kernel(eidx)


def _sc_gather_scatter(y, eidx, e):
    n, d = y.shape
    per_w = e // _NW
    full = per_w // _W
    tail = per_w - full * _W
    assert full % 2 == 0, "double buffering needs an even full-window count"
    rows_per_sub, row_start = _row_split(n)

    @functools.partial(
        pl.kernel,
        out_type=jax.ShapeDtypeStruct((_NC, n, d), jnp.float32),
        mesh=_sc_mesh(),
        scratch_types=[
            pltpu.VMEM((1, _W), jnp.int32),
            pltpu.VMEM((1, _W), jnp.int32),
            pltpu.VMEM((1, _W), jnp.int32),
            pltpu.VMEM((1, _W), jnp.int32),
            pltpu.VMEM((1, _W), jnp.int32),
            pltpu.VMEM((1, _W), jnp.int32),
            pltpu.VMEM((1, 16), jnp.int32),
            pltpu.VMEM((1, 16), jnp.int32),
            pltpu.VMEM((_W, d), jnp.float32),
            pltpu.VMEM((_W, d), jnp.float32),
            pltpu.VMEM_SHARED((n, d), jnp.float32),
            pltpu.SemaphoreType.DMA,
            pltpu.SemaphoreType.DMA,
            pltpu.SemaphoreType.DMA,
            pltpu.SemaphoreType.DMA,
            pltpu.SemaphoreType.DMA,
            pltpu.SemaphoreType.DMA,
        ],
    )
    def gs_kernel(y_hbm, eidx_hbm, out_hbm,
                  sa, sb, da, db, da2, db2, st, dt, rows_a, rows_b, acc,
                  sem_a, sem_b, sem_ia, sem_ib, sem_sa, sem_sb):
        c = lax.axis_index("c")
        s = lax.axis_index("s")
        wid = c * _NS + s
        base_e = wid * per_w

        zero_pat = jnp.zeros((16,), jnp.float32)

        @pl.loop(0, _W)
        def _(i):
            @pl.loop(0, d, step=16)
            def _(j):
                rows_a[i, pl.ds(j, 16)] = zero_pat

        for sub in range(_NS):
            @pl.when(s == sub)
            def _():
                base = row_start[sub]
                nrow = rows_per_sub[sub]
                nfull = nrow // _W
                for k in range(nfull):
                    pltpu.sync_copy(rows_a.at[pl.ds(0, _W)],
                                    acc.at[pl.ds(base + k * _W, _W)])
                rem = nrow - nfull * _W
                if rem:
                    pltpu.sync_copy(rows_a.at[pl.ds(0, rem)],
                                    acc.at[pl.ds(base + nfull * _W, rem)])

        plsc.subcore_barrier()

        def start_idx(win, sref, dref, sem):
            off = base_e + win * _W
            pltpu.async_copy(eidx_hbm.at[pl.ds(off, _W)], sref.at[0], sem)
            pltpu.async_copy(eidx_hbm.at[pl.ds(e + off, _W)],
                             dref.at[0], sem)

        def wait_idx(win, sref, dref, sem):
            off = base_e + win * _W
            pltpu.make_async_copy(eidx_hbm.at[pl.ds(off, _W)],
                                  sref.at[0], sem).wait()
            pltpu.make_async_copy(eidx_hbm.at[pl.ds(e + off, _W)],
                                  dref.at[0], sem).wait()

        start_idx(0, sa, da, sem_ia)
        start_idx(1, sb, db, sem_ib)
        wait_idx(0, sa, da, sem_ia)
        pltpu.async_copy(y_hbm.at[sa.at[0]], rows_a, sem_a)

        def idx_copy(dref, d2ref):
            @pl.loop(0, _W, step=16)
            def _(j):
                d2ref[0, pl.ds(j, 16)] = dref[0, pl.ds(j, 16)]

        @pl.loop(0, full, step=2)
        def _(i):
            pltpu.make_async_copy(y_hbm.at[sa.at[0]], rows_a, sem_a).wait()
            wait_idx(i + 1, sb, db, sem_ib)
            pltpu.async_copy(y_hbm.at[sb.at[0]], rows_b, sem_b)
            idx_copy(da, da2)
            pltpu.async_copy(rows_a, acc.at[da2.at[0]], sem_sa, add=True)

            @pl.when(i + 2 < full)
            def _():
                start_idx(i + 2, sa, da, sem_ia)

            pltpu.make_async_copy(y_hbm.at[sb.at[0]], rows_b, sem_b).wait()
            idx_copy(db, db2)
            pltpu.async_copy(rows_b, acc.at[db2.at[0]], sem_sb, add=True)

            @pl.when(i + 3 < full)
            def _():
                start_idx(i + 3, sb, db, sem_ib)

            pltpu.make_async_copy(rows_a, acc.at[da2.at[0]], sem_sa).wait()

            @pl.when(i + 2 < full)
            def _():
                wait_idx(i + 2, sa, da, sem_ia)
                pltpu.async_copy(y_hbm.at[sa.at[0]], rows_a, sem_a)

            pltpu.make_async_copy(rows_b, acc.at[db2.at[0]], sem_sb).wait()

        if tail:
            off = base_e + full * _W
            pltpu.sync_copy(eidx_hbm.at[pl.ds(off, tail)], st.at[0])
            pltpu.sync_copy(eidx_hbm.at[pl.ds(e + off, tail)], dt.at[0])
            pltpu.sync_copy(y_hbm.at[st.at[0]], rows_a.at[pl.ds(0, tail)])
            pltpu.sync_copy(rows_a.at[pl.ds(0, tail)],
                            acc.at[dt.at[0]], add=True)

        plsc.subcore_barrier()
        for sub in range(_NS):
            @pl.when(s == sub)
            def _():
                base = row_start[sub]
                nrow = rows_per_sub[sub]
                pltpu.sync_copy(acc.at[pl.ds(base, nrow)],
                                out_hbm.at[c].at[pl.ds(base, nrow)])

    return gs_kernel(y, eidx)


def _mm_body(x_ref, w_ref, o_ref):
    o_ref[...] = jnp.dot(x_ref[...], w_ref[...],
                         preferred_element_type=jnp.float32)


def _tc_matmul(x, w):
    n = x.shape[0]
    return pl.pallas_call(
        _mm_body,
        out_shape=jax.ShapeDtypeStruct((n, w.shape[1]), jnp.float32),
    )(x, w)


def _scale_body(d0_ref, d1_ref, xw_ref, y_ref, dinv_ref):
    deg = d0_ref[...] + d1_ref[...] + 1.0
    dinv = lax.rsqrt(deg)
    dinv_ref[...] = dinv
    y_ref[...] = xw_ref[...] * dinv


def _tc_scale(d0, d1, xw):
    n, d = xw.shape
    return pl.pallas_call(
        _scale_body,
        out_shape=(jax.ShapeDtypeStruct((n, d), jnp.float32),
                   jax.ShapeDtypeStruct((n, 1), jnp.float32)),
    )(d0, d1, xw)


def _mid_body(acc_ref, y1_ref, dinv_ref, b1_ref, w2_ref, y2_ref):
    dinv = dinv_ref[...]
    out1 = (acc_ref[0] + acc_ref[1] + y1_ref[...]) * dinv + b1_ref[...]
    h = 0.5 * out1 * (1.0 + lax.erf(out1 * np.float32(0.7071067811865476)))
    y2_ref[...] = jnp.dot(h, w2_ref[...],
                          preferred_element_type=jnp.float32) * dinv


def _tc_mid(acc1, y1, dinv, b1, w2):
    n = y1.shape[0]
    return pl.pallas_call(
        _mid_body,
        out_shape=jax.ShapeDtypeStruct((n, w2.shape[1]), jnp.float32),
    )(acc1, y1, dinv, b1, w2)


def _final_body(acc_ref, y2_ref, dinv_ref, b2_ref, batch_ref, fcw_ref,
                fcb_ref, o_ref):
    n = y2_ref.shape[0]
    out2 = (acc_ref[0] + acc_ref[1] + y2_ref[...]) * dinv_ref[...] \
        + b2_ref[...]
    gid = lax.broadcasted_iota(jnp.int32, (_G, 1), 0)
    seg_t = (batch_ref[...] == gid).astype(jnp.float32)
    dn = (((1,), (0,)), ((), ()))
    psum = lax.dot_general(seg_t, out2, dn,
                           preferred_element_type=jnp.float32)
    ones = jnp.ones((n, 1), jnp.float32)
    cnt = lax.dot_general(seg_t, ones, dn,
                          preferred_element_type=jnp.float32)
    pooled = psum / jnp.maximum(cnt, 1.0)
    o_ref[...] = jnp.dot(pooled, fcw_ref[...],
                         preferred_element_type=jnp.float32) + fcb_ref[...]


def _tc_final(acc2, y2, dinv, b2, batch2d, fcW, fcb):
    return pl.pallas_call(
        _final_body,
        out_shape=jax.ShapeDtypeStruct((_G, fcW.shape[1]), jnp.float32),
    )(acc2, y2, dinv, b2, batch2d, fcW, fcb)


def kernel(x, edge_index, batch, W1, b1, W2, b2, fcW, fcb):
    n = x.shape[0]
    e = edge_index.shape[1]
    assert e % (_NW * 8) == 0, "edge count must split 8-aligned across workers"

    eidx = edge_index.reshape(2 * e)
    batch_row = batch.reshape(1, n)

    xw1 = _tc_matmul(x, W1)
    deg_p = _sc_deg(eidx, e, n)
    d0 = deg_p[0, :n].reshape(n, 1)
    d1 = deg_p[1, :n].reshape(n, 1)
    y1, dinv = _tc_scale(d0, d1, xw1)
    acc1 = _sc_gather_scatter(y1, eidx, e)
    y2 = _tc_mid(acc1, y1, dinv, b1, W2)
    acc2 = _sc_gather_scatter(y2, eidx, e)
    return _tc_final(acc2, y2, dinv, b2, batch_row, fcW, fcb)


# speedup vs baseline: 1.0955x; 1.0514x over previous
import functools

import jax
import jax.numpy as jnp
import numpy as np
from jax import lax
from jax.experimental import pallas as pl
from jax.experimental.pallas import tpu as pltpu
from jax.experimental.pallas import tpu_sc as plsc

_NC = 2
_NS = 16
_NW = _NC * _NS
_W = 128
_G = 64
_NSLOT = 6


def _sc_mesh():
    return plsc.VectorSubcoreMesh(core_axis_name="c", subcore_axis_name="s")


def _row_split(n):
    assert n % 8 == 0
    per = (n // (8 * _NS)) * 8
    rows = [per] * _NS
    rows[-1] += n - per * _NS
    starts = [sum(rows[:i]) for i in range(_NS)]
    return rows, starts


def _sc_deg(eidx, e, n):
    per_w = e // _NW
    full = per_w // _W
    tail = per_w - full * _W
    assert full % _NSLOT == 0, "slot rotation must tile the window count"
    n_pad = ((n + 127) // 128) * 128
    blocks = n_pad // 128
    blk_per = [blocks // _NS + (1 if i < blocks % _NS else 0)
               for i in range(_NS)]
    blk_start = [sum(blk_per[:i]) for i in range(_NS)]

    @functools.partial(
        pl.kernel,
        out_type=jax.ShapeDtypeStruct((_NC, n_pad), jnp.float32),
        mesh=_sc_mesh(),
        scratch_types=(
            [pltpu.VMEM((1, _W), jnp.int32)] * (2 * _NSLOT)
            + [
                pltpu.VMEM((1, 16), jnp.int32),
                pltpu.VMEM((_W,), jnp.float32),
                pltpu.VMEM((_W,), jnp.float32),
                pltpu.VMEM((16,), jnp.float32),
                pltpu.VMEM_SHARED((n_pad,), jnp.float32),
            ]
            + [pltpu.SemaphoreType.DMA] * (2 * _NSLOT)
        ),
    )
    def deg_kernel(dst_hbm, out_hbm, *rest):
        didx = rest[:_NSLOT]
        didx2 = rest[_NSLOT:2 * _NSLOT]
        dtail, zbuf, obuf, otail, acc = rest[2 * _NSLOT:2 * _NSLOT + 5]
        lsem = rest[2 * _NSLOT + 5:3 * _NSLOT + 5]
        ssem = rest[3 * _NSLOT + 5:4 * _NSLOT + 5]
        c = lax.axis_index("c")
        s = lax.axis_index("s")
        wid = c * _NS + s

        zero_pat = jnp.zeros((16,), jnp.float32)
        one_pat = jnp.full((16,), 1.0, jnp.float32)

        @pl.loop(0, _W, step=16)
        def _(i):
            zbuf[pl.ds(i, 16)] = zero_pat
            obuf[pl.ds(i, 16)] = one_pat
        otail[...] = one_pat

        for sub in range(_NS):
            @pl.when(s == sub)
            def _():
                for k in range(blk_per[sub]):
                    off = (blk_start[sub] + k) * 128
                    pltpu.sync_copy(zbuf, acc.at[pl.ds(off, 128)])

        plsc.subcore_barrier()

        def start_load(win, dref, sem):
            pltpu.async_copy(
                dst_hbm.at[pl.ds(e + wid * per_w + win * _W, _W)],
                dref.at[0], sem)

        def wait_load(win, dref, sem):
            pltpu.make_async_copy(
                dst_hbm.at[pl.ds(e + wid * per_w + win * _W, _W)],
                dref.at[0], sem).wait()

        for k in range(_NSLOT):
            start_load(k, didx[k], lsem[k])

        @pl.loop(0, full, step=_NSLOT)
        def _(i):
            for k in range(_NSLOT):
                wait_load(i + k, didx[k], lsem[k])

                @pl.when(i > 0)
                def _():
                    pltpu.make_async_copy(obuf, acc.at[didx2[k].at[0]],
                                          ssem[k]).wait()

                @pl.loop(0, _W, step=16)
                def _(j):
                    didx2[k][0, pl.ds(j, 16)] = didx[k][0, pl.ds(j, 16)]

                @pl.when(i + k + _NSLOT < full)
                def _():
                    start_load(i + k + _NSLOT, didx[k], lsem[k])

                pltpu.async_copy(obuf, acc.at[didx2[k].at[0]], ssem[k],
                                 add=True)

        for k in range(_NSLOT):
            pltpu.make_async_copy(obuf, acc.at[didx2[k].at[0]],
                                  ssem[k]).wait()

        if tail:
            pltpu.sync_copy(
                dst_hbm.at[pl.ds(e + wid * per_w + full * _W, tail)],
                dtail.at[0])
            pltpu.sync_copy(otail, acc.at[dtail.at[0]], add=True)

        plsc.subcore_barrier()
        for sub in range(_NS):
            @pl.when(s == sub)
            def _():
                off = blk_start[sub] * 128
                nrow = blk_per[sub] * 128
                pltpu.sync_copy(acc.at[pl.ds(off, nrow)],
                                out_hbm.at[c].at[pl.ds(off, nrow)])

    return deg_kernel(eidx)


def _sc_gather_scatter(y, eidx, e):
    n, d = y.shape
    per_w = e // _NW
    full = per_w // _W
    tail = per_w - full * _W
    assert full % 2 == 0, "double buffering needs an even full-window count"
    rows_per_sub, row_start = _row_split(n)

    @functools.partial(
        pl.kernel,
        out_type=jax.ShapeDtypeStruct((_NC, n, d), jnp.float32),
        mesh=_sc_mesh(),
        scratch_types=[
            pltpu.VMEM((1, _W), jnp.int32),
            pltpu.VMEM((1, _W), jnp.int32),
            pltpu.VMEM((1, _W), jnp.int32),
            pltpu.VMEM((1, _W), jnp.int32),
            pltpu.VMEM((1, _W), jnp.int32),
            pltpu.VMEM((1, _W), jnp.int32),
            pltpu.VMEM((1, 16), jnp.int32),
            pltpu.VMEM((1, 16), jnp.int32),
            pltpu.VMEM((_W, d), jnp.float32),
            pltpu.VMEM((_W, d), jnp.float32),
            pltpu.VMEM_SHARED((n, d), jnp.float32),
            pltpu.SemaphoreType.DMA,
            pltpu.SemaphoreType.DMA,
            pltpu.SemaphoreType.DMA,
            pltpu.SemaphoreType.DMA,
            pltpu.SemaphoreType.DMA,
            pltpu.SemaphoreType.DMA,
        ],
    )
    def gs_kernel(y_hbm, eidx_hbm, out_hbm,
                  sa, sb, da, db, da2, db2, st, dt, rows_a, rows_b, acc,
                  sem_a, sem_b, sem_ia, sem_ib, sem_sa, sem_sb):
        c = lax.axis_index("c")
        s = lax.axis_index("s")
        wid = c * _NS + s
        base_e = wid * per_w

        zero_pat = jnp.zeros((16,), jnp.float32)

        @pl.loop(0, _W)
        def _(i):
            @pl.loop(0, d, step=16)
            def _(j):
                rows_a[i, pl.ds(j, 16)] = zero_pat

        for sub in range(_NS):
            @pl.when(s == sub)
            def _():
                base = row_start[sub]
                nrow = rows_per_sub[sub]
                nfull = nrow // _W
                for k in range(nfull):
                    pltpu.sync_copy(rows_a.at[pl.ds(0, _W)],
                                    acc.at[pl.ds(base + k * _W, _W)])
                rem = nrow - nfull * _W
                if rem:
                    pltpu.sync_copy(rows_a.at[pl.ds(0, rem)],
                                    acc.at[pl.ds(base + nfull * _W, rem)])

        plsc.subcore_barrier()

        def start_idx(win, sref, dref, sem):
            off = base_e + win * _W
            pltpu.async_copy(eidx_hbm.at[pl.ds(off, _W)], sref.at[0], sem)
            pltpu.async_copy(eidx_hbm.at[pl.ds(e + off, _W)],
                             dref.at[0], sem)

        def wait_idx(win, sref, dref, sem):
            off = base_e + win * _W
            pltpu.make_async_copy(eidx_hbm.at[pl.ds(off, _W)],
                                  sref.at[0], sem).wait()
            pltpu.make_async_copy(eidx_hbm.at[pl.ds(e + off, _W)],
                                  dref.at[0], sem).wait()

        start_idx(0, sa, da, sem_ia)
        start_idx(1, sb, db, sem_ib)
        wait_idx(0, sa, da, sem_ia)
        pltpu.async_copy(y_hbm.at[sa.at[0]], rows_a, sem_a)

        def idx_copy(dref, d2ref):
            @pl.loop(0, _W, step=16)
            def _(j):
                d2ref[0, pl.ds(j, 16)] = dref[0, pl.ds(j, 16)]

        @pl.loop(0, full, step=2)
        def _(i):
            pltpu.make_async_copy(y_hbm.at[sa.at[0]], rows_a, sem_a).wait()
            wait_idx(i + 1, sb, db, sem_ib)
            pltpu.async_copy(y_hbm.at[sb.at[0]], rows_b, sem_b)
            idx_copy(da, da2)
            pltpu.async_copy(rows_a, acc.at[da2.at[0]], sem_sa, add=True)

            @pl.when(i + 2 < full)
            def _():
                start_idx(i + 2, sa, da, sem_ia)

            pltpu.make_async_copy(y_hbm.at[sb.at[0]], rows_b, sem_b).wait()
            idx_copy(db, db2)
            pltpu.async_copy(rows_b, acc.at[db2.at[0]], sem_sb, add=True)

            @pl.when(i + 3 < full)
            def _():
                start_idx(i + 3, sb, db, sem_ib)

            pltpu.make_async_copy(rows_a, acc.at[da2.at[0]], sem_sa).wait()

            @pl.when(i + 2 < full)
            def _():
                wait_idx(i + 2, sa, da, sem_ia)
                pltpu.async_copy(y_hbm.at[sa.at[0]], rows_a, sem_a)

            pltpu.make_async_copy(rows_b, acc.at[db2.at[0]], sem_sb).wait()

        if tail:
            off = base_e + full * _W
            pltpu.sync_copy(eidx_hbm.at[pl.ds(off, tail)], st.at[0])
            pltpu.sync_copy(eidx_hbm.at[pl.ds(e + off, tail)], dt.at[0])
            pltpu.sync_copy(y_hbm.at[st.at[0]], rows_a.at[pl.ds(0, tail)])
            pltpu.sync_copy(rows_a.at[pl.ds(0, tail)],
                            acc.at[dt.at[0]], add=True)

        plsc.subcore_barrier()
        for sub in range(_NS):
            @pl.when(s == sub)
            def _():
                base = row_start[sub]
                nrow = rows_per_sub[sub]
                pltpu.sync_copy(acc.at[pl.ds(base, nrow)],
                                out_hbm.at[c].at[pl.ds(base, nrow)])

    return gs_kernel(y, eidx)


def _mm_body(x_ref, w_ref, o_ref):
    o_ref[...] = jnp.dot(x_ref[...], w_ref[...],
                         preferred_element_type=jnp.float32)


def _tc_matmul(x, w):
    n = x.shape[0]
    return pl.pallas_call(
        _mm_body,
        out_shape=jax.ShapeDtypeStruct((n, w.shape[1]), jnp.float32),
    )(x, w)


def _scale_body(d0_ref, d1_ref, xw_ref, y_ref, dinv_ref):
    deg = d0_ref[...] + d1_ref[...] + 1.0
    dinv = lax.rsqrt(deg)
    dinv_ref[...] = dinv
    y_ref[...] = xw_ref[...] * dinv


def _tc_scale(d0, d1, xw):
    n, d = xw.shape
    return pl.pallas_call(
        _scale_body,
        out_shape=(jax.ShapeDtypeStruct((n, d), jnp.float32),
                   jax.ShapeDtypeStruct((n, 1), jnp.float32)),
    )(d0, d1, xw)


def _mid_body(acc_ref, y1_ref, dinv_ref, b1_ref, w2_ref, y2_ref):
    dinv = dinv_ref[...]
    out1 = (acc_ref[0] + acc_ref[1] + y1_ref[...]) * dinv + b1_ref[...]
    h = 0.5 * out1 * (1.0 + lax.erf(out1 * np.float32(0.7071067811865476)))
    y2_ref[...] = jnp.dot(h, w2_ref[...],
                          preferred_element_type=jnp.float32) * dinv


def _tc_mid(acc1, y1, dinv, b1, w2):
    n = y1.shape[0]
    return pl.pallas_call(
        _mid_body,
        out_shape=jax.ShapeDtypeStruct((n, w2.shape[1]), jnp.float32),
    )(acc1, y1, dinv, b1, w2)


def _final_body(acc_ref, y2_ref, dinv_ref, b2_ref, batch_ref, fcw_ref,
                fcb_ref, o_ref):
    n = y2_ref.shape[0]
    out2 = (acc_ref[0] + acc_ref[1] + y2_ref[...]) * dinv_ref[...] \
        + b2_ref[...]
    gid = lax.broadcasted_iota(jnp.int32, (_G, 1), 0)
    seg_t = (batch_ref[...] == gid).astype(jnp.float32)
    dn = (((1,), (0,)), ((), ()))
    psum = lax.dot_general(seg_t, out2, dn,
                           preferred_element_type=jnp.float32)
    ones = jnp.ones((n, 1), jnp.float32)
    cnt = lax.dot_general(seg_t, ones, dn,
                          preferred_element_type=jnp.float32)
    pooled = psum / jnp.maximum(cnt, 1.0)
    o_ref[...] = jnp.dot(pooled, fcw_ref[...],
                         preferred_element_type=jnp.float32) + fcb_ref[...]


def _tc_final(acc2, y2, dinv, b2, batch2d, fcW, fcb):
    return pl.pallas_call(
        _final_body,
        out_shape=jax.ShapeDtypeStruct((_G, fcW.shape[1]), jnp.float32),
    )(acc2, y2, dinv, b2, batch2d, fcW, fcb)


def kernel(x, edge_index, batch, W1, b1, W2, b2, fcW, fcb):
    n = x.shape[0]
    e = edge_index.shape[1]
    assert e % (_NW * 8) == 0, "edge count must split 8-aligned across workers"

    eidx = edge_index.reshape(2 * e)
    batch_row = batch.reshape(1, n)

    xw1 = _tc_matmul(x, W1)
    deg_p = _sc_deg(eidx, e, n)
    d0 = deg_p[0, :n].reshape(n, 1)
    d1 = deg_p[1, :n].reshape(n, 1)
    y1, dinv = _tc_scale(d0, d1, xw1)
    acc1 = _sc_gather_scatter(y1, eidx, e)
    y2 = _tc_mid(acc1, y1, dinv, b1, W2)
    acc2 = _sc_gather_scatter(y2, eidx, e)
    return _tc_final(acc2, y2, dinv, b2, batch_row, fcW, fcb)
